# Initial kernel scaffold; baseline (speedup 1.0000x reference)
import jax
import jax.numpy as jnp
from jax.experimental import pallas as pl


def kernel(x, adjacency_indices, adjacency_values, cls, W1, W2, Wl, bl):
    raise NotImplementedError("write your pallas kernel here")


# speedup vs baseline: 7.4619x; 7.4619x over previous
import functools

import jax
import jax.numpy as jnp
from jax import lax
from jax.experimental import pallas as pl
from jax.experimental.pallas import tpu as pltpu
from jax.experimental.pallas import tpu_sc as plsc

N = 10000
E = 320000
D = 128

NC = 2
NS = 16
NW = NC * NS
K = 64
NCHUNK = 160
NBLK = 8
NBLOCKS = NCHUNK // NBLK
EPW = K * NCHUNK
NPAD = 10240
NC8 = 1280
ACC_ROWS = 12288
SPT = ACC_ROWS // NS
TPT = SPT // K


def _sc_edge_kernel(x_hbm, iota_hbm, src_hbm, srcc_hbm, dst_hbm, vals_hbm,
                    outp_hbm,
                    src_v, srcc_v, dst_v, vals_v, rows_v, bc_v, it_v, acc_sh,
                    sem):
    cid = lax.axis_index("c")
    sid = lax.axis_index("s")
    wid = sid * NC + cid
    row0 = sid * SPT

    pltpu.sync_copy(iota_hbm.at[pl.ds(sid * TPT, TPT)], it_v)

    def zero_rows(k, carry):
        z = jnp.zeros((16,), jnp.float32)
        for f in range(D // 16):
            rows_v[k, pl.ds(f * 16, 16)] = z
        return carry
    lax.fori_loop(0, K, zero_rows, 0)

    for t in range(TPT):
        pltpu.sync_copy(rows_v, acc_sh.at[it_v.at[t, 0]])

    plsc.subcore_barrier()

    def group_body(i, g, carry):
        vals16 = vals_v[i, pl.ds(g * 16, 16)]
        src16 = src_v[i, 0, pl.ds(g * 16, 16)]
        mod16 = jnp.bitwise_and(src16, 7)
        k0 = g * 16
        for j in range(16):
            val = vals16[j]
            slot = mod16[j]
            for f in range(D // 16):
                sl = pl.ds(f * 16, 16)
                rows_v[k0 + j, sl] = rows_v[k0 + j, sl] * val
                bc_v[k0 + j, sl] = jnp.full(
                    (16,), jnp.where(slot == f, val, 0.0), jnp.float32)
        return carry

    def chunk_body(i, carry):
        pltpu.async_copy(x_hbm.at[src_v.at[i, 0]], rows_v, sem).wait()
        lax.fori_loop(0, K // 16, functools.partial(group_body, i), 0)
        pltpu.sync_copy(rows_v, acc_sh.at[dst_v.at[i, 0]], add=True)
        pltpu.sync_copy(bc_v, acc_sh.at[srcc_v.at[i, 0]], add=True)
        return carry

    def block_body(b, carry):
        c0 = b * NBLK
        pltpu.sync_copy(src_hbm.at[wid, pl.ds(c0, NBLK)], src_v)
        pltpu.sync_copy(srcc_hbm.at[wid, pl.ds(c0, NBLK)], srcc_v)
        pltpu.sync_copy(dst_hbm.at[wid, pl.ds(c0, NBLK)], dst_v)
        pltpu.sync_copy(vals_hbm.at[wid, pl.ds(c0, NBLK)], vals_v)
        lax.fori_loop(0, NBLK, chunk_body, carry)
        return carry

    lax.fori_loop(0, NBLOCKS, block_body, 0)

    plsc.subcore_barrier()

    for t in range(TPT):
        pltpu.sync_copy(acc_sh.at[it_v.at[t, 0]], rows_v)
        pltpu.sync_copy(rows_v, outp_hbm.at[cid, pl.ds(row0 + t * K, K)])


_sc_edge = functools.partial(
    pl.kernel,
    out_type=[
        jax.ShapeDtypeStruct((NC, ACC_ROWS, D), jnp.float32),
    ],
    mesh=plsc.VectorSubcoreMesh(core_axis_name="c", subcore_axis_name="s",
                                num_cores=NC, num_subcores=NS),
    scratch_types=[
        pltpu.VMEM((NBLK, 1, K), jnp.int32),
        pltpu.VMEM((NBLK, 1, K), jnp.int32),
        pltpu.VMEM((NBLK, 1, K), jnp.int32),
        pltpu.VMEM((NBLK, K), jnp.float32),
        pltpu.VMEM((K, D), jnp.float32),
        pltpu.VMEM((K, D), jnp.float32),
        pltpu.VMEM((TPT, 1, K), jnp.int32),
        pltpu.VMEM_SHARED((ACC_ROWS, D), jnp.float32),
        pltpu.SemaphoreType.DMA,
    ],
)(_sc_edge_kernel)


def _tc_finish_kernel(p_ref, c_ref, w1_ref, w2_ref, cls_ref, wl1_ref,
                      wl2_ref, bl_ref, o_ref):
    f32 = jnp.float32
    ps = p_ref[0, :NPAD] + p_ref[1, :NPAD]
    h = lax.dot_general(ps, w1_ref[...], (((1,), (1,)), ((), ())),
                        preferred_element_type=f32)
    r = jnp.maximum(h, 0.0)
    c = c_ref[0] + c_ref[1]
    v = lax.dot_general(c, r, (((1,), (0,)), ((), ())),
                        preferred_element_type=f32)
    z = lax.dot_general(v, w2_ref[...], (((1,), (1,)), ((), ())),
                        preferred_element_type=f32)
    s = 1.0 / (1.0 + jnp.exp(-z))
    out = (lax.dot_general(s, wl1_ref[...], (((1,), (1,)), ((), ())),
                           preferred_element_type=f32)
           + lax.dot_general(cls_ref[...], wl2_ref[...],
                             (((1,), (1,)), ((), ())),
                             preferred_element_type=f32)
           + bl_ref[...])
    o_ref[...] = out


def kernel(x, adjacency_indices, adjacency_values, cls, W1, W2, Wl, bl):
    pad = NW * EPW - E
    pad_idx = jnp.arange(pad, dtype=jnp.int32) % N
    dstf = jnp.concatenate([adjacency_indices[0], pad_idx])
    srcf = jnp.concatenate([adjacency_indices[1], pad_idx])
    dst = dstf.reshape(NW, NCHUNK, 1, K)
    src = srcf.reshape(NW, NCHUNK, 1, K)
    srcc = (NPAD + (srcf >> 3)).reshape(NW, NCHUNK, 1, K)
    vals = jnp.pad(adjacency_values, (0, pad)).reshape(NW, NCHUNK, K)
    iota = jnp.arange(ACC_ROWS, dtype=jnp.int32).reshape(ACC_ROWS // K, 1, K)

    (outp,) = _sc_edge(x, iota, src, srcc, dst, vals)

    c2 = outp[:, NPAD:NPAD + NC8, :].reshape(NC, NPAD, 16)[:, :, 0]
    c2 = c2.reshape(NC, 1, NPAD)

    out = pl.pallas_call(
        _tc_finish_kernel,
        out_shape=jax.ShapeDtypeStruct((1, 4), jnp.float32),
    )(outp, c2, W1, W2, cls, Wl[:, :D], Wl[:, D:], bl[None, :])
    return out


# speedup vs baseline: 10.2112x; 1.3684x over previous
import functools

import jax
import jax.numpy as jnp
from jax import lax
from jax.experimental import pallas as pl
from jax.experimental.pallas import tpu as pltpu
from jax.experimental.pallas import tpu_sc as plsc

N = 10000
E = 320000
D = 128

NC = 2
NS = 16
NW = NC * NS
K = 64
NCHUNK = 160
NBLK = 8
NBLOCKS = NCHUNK // NBLK
EPW = K * NCHUNK
NPAD = 10240
NC8 = 1280
ACC_ROWS = 12288
SPT = ACC_ROWS // NS
TPT = SPT // K


def _sc_edge_kernel(x_hbm, iota_hbm, src_hbm, srcc_hbm, dst_hbm, vals_hbm,
                    outp_hbm,
                    src_v, srcc_v, dst_v, vals_v, rows_v, rows2_v, bc_v, it_v,
                    acc_sh, sem, sem2):
    cid = lax.axis_index("c")
    sid = lax.axis_index("s")
    wid = sid * NC + cid
    row0 = sid * SPT

    pltpu.sync_copy(iota_hbm.at[pl.ds(sid * TPT, TPT)], it_v)

    def zero_rows(k, carry):
        z = jnp.zeros((16,), jnp.float32)
        for f in range(D // 16):
            rows_v[k, pl.ds(f * 16, 16)] = z
        return carry
    lax.fori_loop(0, K, zero_rows, 0)

    for t in range(TPT):
        pltpu.sync_copy(rows_v, acc_sh.at[it_v.at[t, 0]])

    plsc.subcore_barrier()

    def make_group_body(buf):
        def group_body(i, g, carry):
            vals16 = vals_v[i, pl.ds(g * 16, 16)]
            src16 = src_v[i, 0, pl.ds(g * 16, 16)]
            mod16 = jnp.bitwise_and(src16, 7)
            k0 = g * 16
            for j in range(16):
                val = vals16[j]
                slot = mod16[j]
                for f in range(D // 16):
                    sl = pl.ds(f * 16, 16)
                    buf[k0 + j, sl] = buf[k0 + j, sl] * val
                    bc_v[k0 + j, sl] = jnp.full(
                        (16,), jnp.where(slot == f, val, 0.0), jnp.float32)
            return carry
        return group_body

    def scale_and_scatter(i, buf):
        lax.fori_loop(0, K // 16, functools.partial(make_group_body(buf), i),
                      0)
        pltpu.sync_copy(buf, acc_sh.at[dst_v.at[i, 0]], add=True)
        pltpu.sync_copy(bc_v, acc_sh.at[srcc_v.at[i, 0]], add=True)

    def pair_body(h, carry):
        a = 2 * h
        pltpu.make_async_copy(x_hbm.at[src_v.at[a, 0]], rows_v, sem).wait()
        pltpu.async_copy(x_hbm.at[src_v.at[a + 1, 0]], rows2_v, sem2)
        scale_and_scatter(a, rows_v)
        pltpu.make_async_copy(x_hbm.at[src_v.at[a + 1, 0]], rows2_v,
                              sem2).wait()

        @pl.when(a + 2 < NBLK)
        def _():
            pltpu.async_copy(x_hbm.at[src_v.at[a + 2, 0]], rows_v, sem)
        scale_and_scatter(a + 1, rows2_v)
        return carry

    def block_body(b, carry):
        c0 = b * NBLK
        pltpu.sync_copy(src_hbm.at[wid, pl.ds(c0, NBLK)], src_v)
        pltpu.sync_copy(srcc_hbm.at[wid, pl.ds(c0, NBLK)], srcc_v)
        pltpu.sync_copy(dst_hbm.at[wid, pl.ds(c0, NBLK)], dst_v)
        pltpu.sync_copy(vals_hbm.at[wid, pl.ds(c0, NBLK)], vals_v)
        pltpu.async_copy(x_hbm.at[src_v.at[0, 0]], rows_v, sem)
        lax.fori_loop(0, NBLK // 2, pair_body, carry)
        return carry

    lax.fori_loop(0, NBLOCKS, block_body, 0)

    plsc.subcore_barrier()

    for t in range(TPT):
        pltpu.sync_copy(acc_sh.at[it_v.at[t, 0]], rows_v)
        pltpu.sync_copy(rows_v, outp_hbm.at[cid, pl.ds(row0 + t * K, K)])


_sc_edge = functools.partial(
    pl.kernel,
    out_type=[
        jax.ShapeDtypeStruct((NC, ACC_ROWS, D), jnp.float32),
    ],
    mesh=plsc.VectorSubcoreMesh(core_axis_name="c", subcore_axis_name="s",
                                num_cores=NC, num_subcores=NS),
    scratch_types=[
        pltpu.VMEM((NBLK, 1, K), jnp.int32),
        pltpu.VMEM((NBLK, 1, K), jnp.int32),
        pltpu.VMEM((NBLK, 1, K), jnp.int32),
        pltpu.VMEM((NBLK, K), jnp.float32),
        pltpu.VMEM((K, D), jnp.float32),
        pltpu.VMEM((K, D), jnp.float32),
        pltpu.VMEM((K, D), jnp.float32),
        pltpu.VMEM((TPT, 1, K), jnp.int32),
        pltpu.VMEM_SHARED((ACC_ROWS, D), jnp.float32),
        pltpu.SemaphoreType.DMA,
        pltpu.SemaphoreType.DMA,
    ],
)(_sc_edge_kernel)


def _tc_finish_kernel(p_ref, c_ref, w1_ref, w2_ref, cls_ref, wl1_ref,
                      wl2_ref, bl_ref, o_ref):
    f32 = jnp.float32
    ps = p_ref[0, :NPAD] + p_ref[1, :NPAD]
    h = lax.dot_general(ps, w1_ref[...], (((1,), (1,)), ((), ())),
                        preferred_element_type=f32)
    r = jnp.maximum(h, 0.0)
    c = c_ref[0] + c_ref[1]
    v = lax.dot_general(c, r, (((1,), (0,)), ((), ())),
                        preferred_element_type=f32)
    z = lax.dot_general(v, w2_ref[...], (((1,), (1,)), ((), ())),
                        preferred_element_type=f32)
    s = 1.0 / (1.0 + jnp.exp(-z))
    out = (lax.dot_general(s, wl1_ref[...], (((1,), (1,)), ((), ())),
                           preferred_element_type=f32)
           + lax.dot_general(cls_ref[...], wl2_ref[...],
                             (((1,), (1,)), ((), ())),
                             preferred_element_type=f32)
           + bl_ref[...])
    o_ref[...] = out


def kernel(x, adjacency_indices, adjacency_values, cls, W1, W2, Wl, bl):
    pad = NW * EPW - E
    pad_idx = jnp.arange(pad, dtype=jnp.int32) % N
    dstf = jnp.concatenate([adjacency_indices[0], pad_idx])
    srcf = jnp.concatenate([adjacency_indices[1], pad_idx])
    dst = dstf.reshape(NW, NCHUNK, 1, K)
    src = srcf.reshape(NW, NCHUNK, 1, K)
    srcc = (NPAD + (srcf >> 3)).reshape(NW, NCHUNK, 1, K)
    vals = jnp.pad(adjacency_values, (0, pad)).reshape(NW, NCHUNK, K)
    iota = jnp.arange(ACC_ROWS, dtype=jnp.int32).reshape(ACC_ROWS // K, 1, K)

    (outp,) = _sc_edge(x, iota, src, srcc, dst, vals)

    c2 = outp[:, NPAD:NPAD + NC8, :].reshape(NC, NPAD, 16)[:, :, 0]
    c2 = c2.reshape(NC, 1, NPAD)

    out = pl.pallas_call(
        _tc_finish_kernel,
        out_shape=jax.ShapeDtypeStruct((1, 4), jnp.float32),
    )(outp, c2, W1, W2, cls, Wl[:, :D], Wl[:, D:], bl[None, :])
    return out
